# baseline (speedup 1.0000x reference)
import functools

import jax
import jax.numpy as jnp
from jax import lax
from jax.experimental import pallas as pl
from jax.experimental.pallas import tpu as pltpu
from jax.experimental.pallas import tpu_sc as plsc

EMBED = 64
SCALE = float(EMBED) ** 0.5
LANES = 16

_info = plsc.get_sparse_core_info()
NC, NS = _info.num_cores, _info.num_subcores
NW = NC * NS

CHUNK = 512
SUB = 128
NSUB = CHUNK // SUB


def _body(table_hbm, idx_hbm, out_hbm, idx_v, rows_v, sem):
    wid = lax.axis_index("s") * NC + lax.axis_index("c")
    n_rows_total = out_hbm.shape[0]
    per_w = n_rows_total // NW
    n_chunks = per_w // CHUNK
    base_row = wid * per_w

    def chunk_body(c, _):
        row0 = base_row + c * CHUNK
        pltpu.sync_copy(idx_hbm.at[pl.ds(row0, CHUNK)], idx_v)

        handles = []
        for j in range(NSUB):
            handles.append(
                pltpu.async_copy(
                    table_hbm.at[idx_v.at[pl.ds(j * SUB, SUB)]],
                    rows_v.at[pl.ds(j * SUB, SUB)],
                    sem,
                )
            )
        for h in handles:
            h.wait()

        def group_body(g, _):
            iv = idx_v[pl.ds(g * LANES, LANES)]
            m16 = jnp.where(iv != 0, jnp.float32(SCALE), jnp.float32(0.0))
            for lane in range(LANES):
                r = g * LANES + lane
                m = jnp.full((LANES,), m16[lane])
                for p in range(EMBED // LANES):
                    sl = pl.ds(p * LANES, LANES)
                    rows_v[r, sl] = rows_v[r, sl] * m
            return _

        lax.fori_loop(0, CHUNK // LANES, group_body, None)

        pltpu.sync_copy(rows_v, out_hbm.at[pl.ds(row0, CHUNK)])
        return _

    lax.fori_loop(0, n_chunks, chunk_body, None)


def kernel(inputs, shared_weights):
    b, s = inputs.shape
    n = b * s
    idx_flat = inputs.reshape(n)

    mesh = plsc.VectorSubcoreMesh(core_axis_name="c", subcore_axis_name="s")
    out = pl.kernel(
        _body,
        out_type=jax.ShapeDtypeStruct((n, EMBED), jnp.float32),
        mesh=mesh,
        compiler_params=pltpu.CompilerParams(use_tc_tiling_on_sc=False),
        scratch_types=[
            pltpu.VMEM((CHUNK,), jnp.int32),
            pltpu.VMEM((CHUNK, EMBED), jnp.float32),
            pltpu.SemaphoreType.DMA,
        ],
    )(shared_weights, idx_flat)
    return out.reshape(b, s, EMBED)
